# baseline (speedup 1.0000x reference)
import functools

import jax
import jax.numpy as jnp
from jax import lax
from jax.experimental import pallas as pl
from jax.experimental.pallas import tpu as pltpu
from jax.experimental.pallas import tpu_sc as plsc

N, E, F, D, OUT, G = 10000, 320000, 128, 256, 40, 64
NUM_LAYERS = 4
H = 128
NC, NS = 2, 16
CH = 64
EPT = 20480
NCHUNK = EPT // CH
E_PAD = EPT * NS
GCH = NCHUNK // 5
NGRP = NCHUNK // GCH
NB = 4
N_ACC = 10240
ZROWS = N_ACC // NS
BR = 1000
NBLK = N // BR
EPS = 1e-5

@functools.cache
def _sc_mesh():
    return plsc.VectorSubcoreMesh(core_axis_name="c", subcore_axis_name="s",
                                  num_cores=NC, num_subcores=NS)


HW = 16
ZCH = 64


def _sc_deg_body(col3d, ones_hbm, zeros_hbm, out,
                 col_v, ones_v, zb_v, acc):
    cid = lax.axis_index("c")
    sid = lax.axis_index("s")
    pltpu.sync_copy(zeros_hbm, zb_v)
    pltpu.sync_copy(ones_hbm, ones_v)

    def zchunk(t, carry):
        pltpu.sync_copy(zb_v, acc.at[pl.ds(sid * ZROWS + t * ZCH, ZCH)])
        return carry

    lax.fori_loop(0, ZROWS // ZCH, zchunk, 0)
    plsc.subcore_barrier()
    base = cid * (GCH // 2)

    def group(g, carry):
        pltpu.sync_copy(col3d.at[sid, pl.ds(g * GCH, GCH)], col_v)

        def step(k, carry2):
            pltpu.sync_copy(ones_v, acc.at[col_v.at[base + k]], add=True)
            return carry2

        lax.fori_loop(0, GCH // 2, step, 0)
        return carry

    lax.fori_loop(0, NGRP, group, 0)
    plsc.subcore_barrier()
    pltpu.sync_copy(acc.at[pl.ds(sid * ZROWS, ZROWS)],
                    out.at[cid, pl.ds(sid * ZROWS, ZROWS)])


@functools.cache
def _sc_deg():
    return pl.kernel(
        _sc_deg_body,
        out_type=jax.ShapeDtypeStruct((NC, N_ACC, H), jnp.float32),
        mesh=_sc_mesh(),
        scratch_types=[
            pltpu.VMEM((GCH, CH), jnp.int32),
            pltpu.VMEM((CH, H), jnp.float32),
            pltpu.VMEM((ZCH, H), jnp.float32),
            pltpu.VMEM_SHARED((N_ACC, H), jnp.float32),
        ],
    )


def _sc_scatter_body(t_lo, t_hi, row3d, col3d, zrows_hbm, s_lo, s_hi,
                     row_v, col_v, acc, ssem, *rest):
    cid = lax.axis_index("c")
    sid = lax.axis_index("s")
    bufs = rest[:NB]
    gsems = rest[NB:]
    zb_v = bufs[0]
    pltpu.sync_copy(zrows_hbm, zb_v)

    def zchunk(t, carry):
        pltpu.sync_copy(zb_v, acc.at[pl.ds(sid * ZROWS + t * ZCH, ZCH)])
        return carry

    lax.fori_loop(0, ZROWS // ZCH, zchunk, 0)
    plsc.subcore_barrier()

    def run(table):
        def group(g, carry):
            pltpu.sync_copy(row3d.at[sid, pl.ds(g * GCH, GCH)], row_v)
            pltpu.sync_copy(col3d.at[sid, pl.ds(g * GCH, GCH)], col_v)
            for b in range(NB - 1):
                pltpu.async_copy(table.at[row_v.at[b]], bufs[b], gsems[b])

            def quad(m, carry2):
                for b in range(NB):
                    jj = NB * m + b
                    nxt = jj + NB - 1
                    bn = (b + NB - 1) % NB
                    pltpu.make_async_copy(table.at[row_v.at[jj]], bufs[b],
                                          gsems[b]).wait()
                    pltpu.async_copy(bufs[b], acc.at[col_v.at[jj]], ssem,
                                     add=True)

                    @pl.when(jnp.logical_and(jj >= 1, nxt <= GCH - 1))
                    def _():
                        pltpu.make_async_copy(
                            bufs[0], acc.at[col_v.at[0]], ssem).wait()

                    @pl.when(nxt <= GCH - 1)
                    def _():
                        pltpu.async_copy(table.at[row_v.at[nxt]], bufs[bn],
                                         gsems[bn])
                return carry2

            lax.fori_loop(0, GCH // NB, quad, 0)
            for _ in range(NB):
                pltpu.make_async_copy(bufs[0], acc.at[col_v.at[0]],
                                      ssem).wait()
            return carry

        lax.fori_loop(0, NGRP, group, 0)

    @pl.when(cid == 0)
    def _():
        run(t_lo)

    @pl.when(cid == 1)
    def _():
        run(t_hi)

    plsc.subcore_barrier()

    @pl.when(cid == 0)
    def _():
        pltpu.sync_copy(acc.at[pl.ds(sid * ZROWS, ZROWS)],
                        s_lo.at[pl.ds(sid * ZROWS, ZROWS)])

    @pl.when(cid == 1)
    def _():
        pltpu.sync_copy(acc.at[pl.ds(sid * ZROWS, ZROWS)],
                        s_hi.at[pl.ds(sid * ZROWS, ZROWS)])


@functools.cache
def _sc_scatter():
    return pl.kernel(
        _sc_scatter_body,
        out_type=(jax.ShapeDtypeStruct((N_ACC, H), jnp.float32),
                  jax.ShapeDtypeStruct((N_ACC, H), jnp.float32)),
        mesh=_sc_mesh(),
        scratch_types=(
            [pltpu.VMEM((GCH, CH), jnp.int32),
             pltpu.VMEM((GCH, CH), jnp.int32),
             pltpu.VMEM_SHARED((N_ACC, H), jnp.float32),
             pltpu.SemaphoreType.DMA]
            + [pltpu.VMEM((CH, H), jnp.float32)] * NB
            + [pltpu.SemaphoreType.DMA] * NB),
    )


def _dot(a, b):
    return jnp.dot(a, b, preferred_element_type=jnp.float32)


def _prep_body(w2_ref, gw_ref, b2_ref, w12_ref, b12_ref):
    w12_ref[0] = _dot(w2_ref[0], gw_ref[0])
    b12_ref[0] = _dot(b2_ref[0], gw_ref[0])


def _prep(w2s, gws, b2s):
    return pl.pallas_call(
        _prep_body,
        grid=(NUM_LAYERS,),
        in_specs=[
            pl.BlockSpec((1, D, D), lambda i: (i, 0, 0)),
            pl.BlockSpec((1, D, D), lambda i: (i, 0, 0)),
            pl.BlockSpec((1, 8, D), lambda i: (i, 0, 0)),
        ],
        out_specs=(pl.BlockSpec((1, D, D), lambda i: (i, 0, 0)),
                   pl.BlockSpec((1, 8, D), lambda i: (i, 0, 0))),
        out_shape=(jax.ShapeDtypeStruct((NUM_LAYERS, D, D), jnp.float32),
                   jax.ShapeDtypeStruct((NUM_LAYERS, 8, D), jnp.float32)),
    )(w2s, gws, b2s)


def _stats_rows(h):
    s = jnp.sum(h, axis=0)
    q = jnp.sum(h * h, axis=0)
    return jnp.concatenate(
        [s[None], q[None], jnp.zeros((6, s.shape[0]), jnp.float32)], axis=0)


def _dinv_body(degp_ref, dinv_ref):
    dinv_ref[...] = lax.rsqrt(degp_ref[0] + degp_ref[1] + 1.0)


def _dinv_call(degp):
    return pl.pallas_call(
        _dinv_body,
        grid=(NBLK,),
        in_specs=[pl.BlockSpec((NC, BR, H), lambda i: (0, i, 0))],
        out_specs=pl.BlockSpec((BR, H), lambda i: (i, 0)),
        out_shape=jax.ShapeDtypeStruct((N, H), jnp.float32),
    )(degp)


def _a0_body(x_ref, w1_ref, batch_ref,
             u_ref, part_ref, pooled_ref):
    i = pl.program_id(0)
    xb = x_ref[...]
    u = _dot(xb, w1_ref[...])
    u_ref[...] = u
    part_ref[0] = _stats_rows(u)
    b = batch_ref[0, 0]
    oh = (lax.broadcasted_iota(jnp.int32, (G, BR), 0) == b[None, :]
          ).astype(jnp.float32)

    @pl.when(i == 0)
    def _():
        pooled_ref[...] = jnp.zeros((G, F), jnp.float32)

    pooled_ref[...] += _dot(oh, xb)


def _a0(x, w1, batch3):
    return pl.pallas_call(
        _a0_body,
        grid=(NBLK,),
        in_specs=[
            pl.BlockSpec((BR, F), lambda i: (i, 0)),
            pl.BlockSpec((F, D), lambda i: (0, 0)),
            pl.BlockSpec((1, 1, BR), lambda i: (i, 0, 0)),
        ],
        out_specs=(pl.BlockSpec((BR, D), lambda i: (i, 0)),
                   pl.BlockSpec((1, 8, D), lambda i: (i, 0, 0)),
                   pl.BlockSpec((G, F), lambda i: (0, 0))),
        out_shape=(jax.ShapeDtypeStruct((N, D), jnp.float32),
                   jax.ShapeDtypeStruct((NBLK, 8, D), jnp.float32),
                   jax.ShapeDtypeStruct((G, F), jnp.float32)),
    )(x, w1, batch3)


def _bn_coeffs(part, g, b):
    m = jnp.sum(part[:, 0, :], axis=0) * (1.0 / N)
    ex2 = jnp.sum(part[:, 1, :], axis=0) * (1.0 / N)
    v = ex2 - m * m
    scale = lax.rsqrt(v + EPS) * g
    return scale, b - m * scale


def _b_body(u_ref, part_ref, g1_ref, b1_ref, w12_ref, b12_ref, dinv_ref,
            tlo_ref, thi_ref):
    scale, shift = _bn_coeffs(part_ref[...], g1_ref[0], b1_ref[0])
    t = jnp.maximum(u_ref[...] * scale + shift, 0.0)
    hm = _dot(t, w12_ref[...]) + b12_ref[0]
    dv = dinv_ref[...]
    tlo_ref[...] = hm[:, :H] * dv
    thi_ref[...] = hm[:, H:] * dv


def _b_call(u, part, g1, b1, w12, b12, dinv):
    return pl.pallas_call(
        _b_body,
        grid=(NBLK,),
        in_specs=[
            pl.BlockSpec((BR, D), lambda i: (i, 0)),
            pl.BlockSpec((NBLK, 8, D), lambda i: (0, 0, 0)),
            pl.BlockSpec((8, D), lambda i: (0, 0)),
            pl.BlockSpec((8, D), lambda i: (0, 0)),
            pl.BlockSpec((D, D), lambda i: (0, 0)),
            pl.BlockSpec((8, D), lambda i: (0, 0)),
            pl.BlockSpec((BR, H), lambda i: (i, 0)),
        ],
        out_specs=(pl.BlockSpec((BR, H), lambda i: (i, 0)),
                   pl.BlockSpec((BR, H), lambda i: (i, 0))),
        out_shape=(jax.ShapeDtypeStruct((N, H), jnp.float32),
                   jax.ShapeDtypeStruct((N, H), jnp.float32)),
    )(u, part, g1, b1, w12, b12, dinv)


def _c_body(slo_ref, shi_ref, tlo_ref, thi_ref, dinv_ref,
            alo_ref, ahi_ref, part_ref):
    dv = dinv_ref[...]
    alo = dv * (slo_ref[...] + tlo_ref[...])
    ahi = dv * (shi_ref[...] + thi_ref[...])
    alo_ref[...] = alo
    ahi_ref[...] = ahi
    s = jnp.concatenate([jnp.sum(alo, 0), jnp.sum(ahi, 0)])
    q = jnp.concatenate([jnp.sum(alo * alo, 0), jnp.sum(ahi * ahi, 0)])
    part_ref[0] = jnp.concatenate(
        [s[None], q[None], jnp.zeros((6, D), jnp.float32)], axis=0)


def _c_call(slo, shi, tlo, thi, dinv):
    bs = pl.BlockSpec((BR, H), lambda i: (i, 0))
    return pl.pallas_call(
        _c_body,
        grid=(NBLK,),
        in_specs=[bs, bs, bs, bs, bs],
        out_specs=(bs, bs, pl.BlockSpec((1, 8, D), lambda i: (i, 0, 0))),
        out_shape=(jax.ShapeDtypeStruct((N, H), jnp.float32),
                   jax.ShapeDtypeStruct((N, H), jnp.float32),
                   jax.ShapeDtypeStruct((NBLK, 8, D), jnp.float32)),
    )(slo, shi, tlo, thi, dinv)


def _da_body(alo_ref, ahi_ref, part_ref, g2_ref, b2_ref, batch_ref, w1n_ref,
             pooled_ref, u_ref, parta_ref):
    i = pl.program_id(0)
    scale, shift = _bn_coeffs(part_ref[...], g2_ref[0], b2_ref[0])
    hlo = jnp.maximum(alo_ref[...] * scale[:H] + shift[:H], 0.0)
    hhi = jnp.maximum(ahi_ref[...] * scale[H:] + shift[H:], 0.0)
    b = batch_ref[0, 0]
    oh = (lax.broadcasted_iota(jnp.int32, (G, BR), 0) == b[None, :]
          ).astype(jnp.float32)

    @pl.when(i == 0)
    def _():
        pooled_ref[...] = jnp.zeros((G, D), jnp.float32)

    pooled_ref[...] += jnp.concatenate([_dot(oh, hlo), _dot(oh, hhi)], axis=1)
    if w1n_ref is not None:
        wn = w1n_ref[...]
        u = _dot(hlo, wn[:H, :]) + _dot(hhi, wn[H:, :])
        u_ref[...] = u
        parta_ref[0] = _stats_rows(u)


def _da_call(alo, ahi, part, g2, b2, batch3, w1n):
    bs = pl.BlockSpec((BR, H), lambda i: (i, 0))
    last = w1n is None
    in_specs = [
        bs, bs,
        pl.BlockSpec((NBLK, 8, D), lambda i: (0, 0, 0)),
        pl.BlockSpec((8, D), lambda i: (0, 0)),
        pl.BlockSpec((8, D), lambda i: (0, 0)),
        pl.BlockSpec((1, 1, BR), lambda i: (i, 0, 0)),
    ]
    args = [alo, ahi, part, g2, b2, batch3]
    out_specs = [pl.BlockSpec((G, D), lambda i: (0, 0))]
    out_shape = [jax.ShapeDtypeStruct((G, D), jnp.float32)]
    if last:
        body = functools.partial(_da_body_last)
        return pl.pallas_call(
            body, grid=(NBLK,), in_specs=in_specs,
            out_specs=out_specs[0], out_shape=out_shape[0])(*args)
    in_specs.append(pl.BlockSpec((D, D), lambda i: (0, 0)))
    args.append(w1n)
    out_specs += [pl.BlockSpec((BR, D), lambda i: (i, 0)),
                  pl.BlockSpec((1, 8, D), lambda i: (i, 0, 0))]
    out_shape += [jax.ShapeDtypeStruct((N, D), jnp.float32),
                  jax.ShapeDtypeStruct((NBLK, 8, D), jnp.float32)]
    return pl.pallas_call(
        _da_body, grid=(NBLK,), in_specs=in_specs,
        out_specs=tuple(out_specs), out_shape=tuple(out_shape))(*args)


def _da_body_last(alo_ref, ahi_ref, part_ref, g2_ref, b2_ref, batch_ref,
                  pooled_ref):
    _da_body(alo_ref, ahi_ref, part_ref, g2_ref, b2_ref, batch_ref, None,
             pooled_ref, None, None)


def _fc_body(p0_ref, p1_ref, p2_ref, p3_ref, p4_ref,
             w0_ref, w1_ref, w2_ref, w3_ref, w4_ref, b_ref, o_ref):
    acc = _dot(p0_ref[...], w0_ref[...])
    acc += _dot(p1_ref[...], w1_ref[...])
    acc += _dot(p2_ref[...], w2_ref[...])
    acc += _dot(p3_ref[...], w3_ref[...])
    acc += _dot(p4_ref[...], w4_ref[...])
    o_ref[...] = acc + b_ref[0]


def _fc_call(pools, ws, bsum):
    args = list(pools) + list(ws) + [bsum]
    return pl.pallas_call(
        _fc_body,
        in_specs=[pl.BlockSpec(a.shape, lambda: tuple(0 for _ in a.shape))
                  for a in args],
        out_specs=pl.BlockSpec((G, 128), lambda: (0, 0)),
        out_shape=jax.ShapeDtypeStruct((G, 128), jnp.float32),
    )(*args)


def kernel(x, params, edge_index, batch):
    f32 = jnp.float32
    row = edge_index[0].astype(jnp.int32)
    col = edge_index[1].astype(jnp.int32)
    npad = E_PAD - E
    pad_r = (jnp.arange(npad, dtype=jnp.int32) * 97) % N
    pad_c = N + (jnp.arange(npad, dtype=jnp.int32) % CH)
    row3d = jnp.concatenate([row, pad_r]).reshape(NS, NCHUNK, CH)
    col3d = jnp.concatenate([col, pad_c]).reshape(NS, NCHUNK, CH)
    batch3 = batch.astype(jnp.int32).reshape(NBLK, 1, BR)
    zrows = jnp.zeros((ZCH, H), f32)

    def pad8(v):
        return jnp.broadcast_to(v[None, :], (8, v.shape[0]))

    w2s = jnp.stack([params[f"l2W{i}"] for i in range(NUM_LAYERS)])
    gws = jnp.stack([params[f"gW{i}"] for i in range(NUM_LAYERS)])
    b2s = jnp.stack([pad8(params[f"l2b{i}"]) for i in range(NUM_LAYERS)])
    w12s, b12s = _prep(w2s, gws, b2s)

    ones = jnp.ones((CH, H), f32)
    degp = _sc_deg()(col3d, ones, zrows)

    u, part, pooled_x = _a0(x, params["l1W0"], batch3)
    dinv = _dinv_call(degp)
    pools = [pooled_x]
    for i in range(NUM_LAYERS):
        tlo, thi = _b_call(u, part, pad8(params[f"bn1g{i}"]),
                           pad8(params[f"bn1b{i}"]), w12s[i], b12s[i], dinv)
        slo, shi = _sc_scatter()(tlo, thi, row3d, col3d, zrows)
        alo, ahi, partc = _c_call(slo, shi, tlo, thi, dinv)
        g2 = pad8(params[f"bng{i}"])
        b2 = pad8(params[f"bnb{i}"])
        if i < NUM_LAYERS - 1:
            pooled_i, u, part = _da_call(alo, ahi, partc, g2, b2, batch3,
                                         params[f"l1W{i + 1}"])
        else:
            pooled_i = _da_call(alo, ahi, partc, g2, b2, batch3, None)
        pools.append(pooled_i)

    ws = []
    for i in range(NUM_LAYERS + 1):
        w = params[f"fcW{i}"]
        ws.append(jnp.zeros((w.shape[0], 128), f32).at[:, :OUT].set(w))
    bsum = sum(params[f"fcb{i}"] for i in range(NUM_LAYERS + 1))
    bpad = pad8(jnp.zeros((128,), f32).at[:OUT].set(bsum))
    out = _fc_call(pools, ws, bpad)
    return out[:, :OUT]


# speedup vs baseline: 1.0277x; 1.0277x over previous
import functools

import jax
import jax.numpy as jnp
from jax import lax
from jax.experimental import pallas as pl
from jax.experimental.pallas import tpu as pltpu
from jax.experimental.pallas import tpu_sc as plsc

N, E, F, D, OUT, G = 10000, 320000, 128, 256, 40, 64
NUM_LAYERS = 4
H = 128
NC, NS = 2, 16
CH = 64
EPT = 20480
NCHUNK = EPT // CH
E_PAD = EPT * NS
GCH = NCHUNK // 5
NGRP = NCHUNK // GCH
NB = 4
N_ACC = 10240
ZROWS = N_ACC // NS
BR = 1000
NBLK = N // BR
EPS = 1e-5

@functools.cache
def _sc_mesh():
    return plsc.VectorSubcoreMesh(core_axis_name="c", subcore_axis_name="s",
                                  num_cores=NC, num_subcores=NS)


HW = 16
ZCH = 64


DEGW = HW


def _sc_deg_body(col3d, ones_hbm, zeros_hbm, out,
                 col_v, ones_v, zb_v, acc):
    cid = lax.axis_index("c")
    sid = lax.axis_index("s")
    pltpu.sync_copy(zeros_hbm, zb_v)
    pltpu.sync_copy(ones_hbm, ones_v)

    def zchunk(t, carry):
        pltpu.sync_copy(zb_v, acc.at[pl.ds(sid * ZROWS + t * ZCH, ZCH)])
        return carry

    lax.fori_loop(0, ZROWS // ZCH, zchunk, 0)
    plsc.subcore_barrier()
    base = cid * (GCH // 2)

    def group(g, carry):
        pltpu.sync_copy(col3d.at[sid, pl.ds(g * GCH, GCH)], col_v)

        def step(k, carry2):
            pltpu.sync_copy(ones_v, acc.at[col_v.at[base + k]], add=True)
            return carry2

        lax.fori_loop(0, GCH // 2, step, 0)
        return carry

    lax.fori_loop(0, NGRP, group, 0)
    plsc.subcore_barrier()
    pltpu.sync_copy(acc.at[pl.ds(sid * ZROWS, ZROWS)],
                    out.at[cid, pl.ds(sid * ZROWS, ZROWS)])


@functools.cache
def _sc_deg():
    return pl.kernel(
        _sc_deg_body,
        out_type=jax.ShapeDtypeStruct((NC, N_ACC, DEGW), jnp.float32),
        mesh=_sc_mesh(),
        scratch_types=[
            pltpu.VMEM((GCH, CH), jnp.int32),
            pltpu.VMEM((CH, DEGW), jnp.float32),
            pltpu.VMEM((ZCH, DEGW), jnp.float32),
            pltpu.VMEM_SHARED((N_ACC, DEGW), jnp.float32),
        ],
    )


def _sc_scatter_body(t_lo, t_hi, row3d, col3d, zrows_hbm, s_lo, s_hi,
                     row_v, col_v, acc, ssem, *rest):
    cid = lax.axis_index("c")
    sid = lax.axis_index("s")
    bufs = rest[:NB]
    gsems = rest[NB:]
    zb_v = bufs[0]
    pltpu.sync_copy(zrows_hbm, zb_v)

    def zchunk(t, carry):
        pltpu.sync_copy(zb_v, acc.at[pl.ds(sid * ZROWS + t * ZCH, ZCH)])
        return carry

    lax.fori_loop(0, ZROWS // ZCH, zchunk, 0)
    plsc.subcore_barrier()

    def run(table):
        def group(g, carry):
            pltpu.sync_copy(row3d.at[sid, pl.ds(g * GCH, GCH)], row_v)
            pltpu.sync_copy(col3d.at[sid, pl.ds(g * GCH, GCH)], col_v)
            for b in range(NB - 1):
                pltpu.async_copy(table.at[row_v.at[b]], bufs[b], gsems[b])

            def quad(m, carry2):
                for b in range(NB):
                    jj = NB * m + b
                    nxt = jj + NB - 1
                    bn = (b + NB - 1) % NB
                    pltpu.make_async_copy(table.at[row_v.at[jj]], bufs[b],
                                          gsems[b]).wait()
                    pltpu.async_copy(bufs[b], acc.at[col_v.at[jj]], ssem,
                                     add=True)

                    @pl.when(jnp.logical_and(jj >= 1, nxt <= GCH - 1))
                    def _():
                        pltpu.make_async_copy(
                            bufs[0], acc.at[col_v.at[0]], ssem).wait()

                    @pl.when(nxt <= GCH - 1)
                    def _():
                        pltpu.async_copy(table.at[row_v.at[nxt]], bufs[bn],
                                         gsems[bn])
                return carry2

            lax.fori_loop(0, GCH // NB, quad, 0)
            for _ in range(NB):
                pltpu.make_async_copy(bufs[0], acc.at[col_v.at[0]],
                                      ssem).wait()
            return carry

        lax.fori_loop(0, NGRP, group, 0)

    @pl.when(cid == 0)
    def _():
        run(t_lo)

    @pl.when(cid == 1)
    def _():
        run(t_hi)

    plsc.subcore_barrier()

    @pl.when(cid == 0)
    def _():
        pltpu.sync_copy(acc.at[pl.ds(sid * ZROWS, ZROWS)],
                        s_lo.at[pl.ds(sid * ZROWS, ZROWS)])

    @pl.when(cid == 1)
    def _():
        pltpu.sync_copy(acc.at[pl.ds(sid * ZROWS, ZROWS)],
                        s_hi.at[pl.ds(sid * ZROWS, ZROWS)])


@functools.cache
def _sc_scatter():
    return pl.kernel(
        _sc_scatter_body,
        out_type=(jax.ShapeDtypeStruct((N_ACC, H), jnp.float32),
                  jax.ShapeDtypeStruct((N_ACC, H), jnp.float32)),
        mesh=_sc_mesh(),
        scratch_types=(
            [pltpu.VMEM((GCH, CH), jnp.int32),
             pltpu.VMEM((GCH, CH), jnp.int32),
             pltpu.VMEM_SHARED((N_ACC, H), jnp.float32),
             pltpu.SemaphoreType.DMA]
            + [pltpu.VMEM((CH, H), jnp.float32)] * NB
            + [pltpu.SemaphoreType.DMA] * NB),
    )


def _dot(a, b):
    return jnp.dot(a, b, preferred_element_type=jnp.float32)


def _prep_body(w2_ref, gw_ref, b2_ref, w12_ref, b12_ref):
    w12_ref[0] = _dot(w2_ref[0], gw_ref[0])
    b12_ref[0] = _dot(b2_ref[0], gw_ref[0])


def _prep(w2s, gws, b2s):
    return pl.pallas_call(
        _prep_body,
        grid=(NUM_LAYERS,),
        in_specs=[
            pl.BlockSpec((1, D, D), lambda i: (i, 0, 0)),
            pl.BlockSpec((1, D, D), lambda i: (i, 0, 0)),
            pl.BlockSpec((1, 8, D), lambda i: (i, 0, 0)),
        ],
        out_specs=(pl.BlockSpec((1, D, D), lambda i: (i, 0, 0)),
                   pl.BlockSpec((1, 8, D), lambda i: (i, 0, 0))),
        out_shape=(jax.ShapeDtypeStruct((NUM_LAYERS, D, D), jnp.float32),
                   jax.ShapeDtypeStruct((NUM_LAYERS, 8, D), jnp.float32)),
    )(w2s, gws, b2s)


def _stats_rows(h):
    s = jnp.sum(h, axis=0)
    q = jnp.sum(h * h, axis=0)
    return jnp.concatenate(
        [s[None], q[None], jnp.zeros((6, s.shape[0]), jnp.float32)], axis=0)


def _dinv_body(degp_ref, dinv_ref):
    dinv_ref[...] = lax.rsqrt(degp_ref[0] + degp_ref[1] + 1.0)


def _dinv_call(degp):
    return pl.pallas_call(
        _dinv_body,
        grid=(NBLK,),
        in_specs=[pl.BlockSpec((NC, BR, H), lambda i: (0, i, 0))],
        out_specs=pl.BlockSpec((BR, H), lambda i: (i, 0)),
        out_shape=jax.ShapeDtypeStruct((N, H), jnp.float32),
    )(degp)


def _a0_body(x_ref, w1_ref, batch_ref,
             u_ref, part_ref, pooled_ref):
    i = pl.program_id(0)
    xb = x_ref[...]
    u = _dot(xb, w1_ref[...])
    u_ref[...] = u
    part_ref[0] = _stats_rows(u)
    b = batch_ref[0, 0]
    oh = (lax.broadcasted_iota(jnp.int32, (G, BR), 0) == b[None, :]
          ).astype(jnp.float32)

    @pl.when(i == 0)
    def _():
        pooled_ref[...] = jnp.zeros((G, F), jnp.float32)

    pooled_ref[...] += _dot(oh, xb)


def _a0(x, w1, batch3):
    return pl.pallas_call(
        _a0_body,
        grid=(NBLK,),
        in_specs=[
            pl.BlockSpec((BR, F), lambda i: (i, 0)),
            pl.BlockSpec((F, D), lambda i: (0, 0)),
            pl.BlockSpec((1, 1, BR), lambda i: (i, 0, 0)),
        ],
        out_specs=(pl.BlockSpec((BR, D), lambda i: (i, 0)),
                   pl.BlockSpec((1, 8, D), lambda i: (i, 0, 0)),
                   pl.BlockSpec((G, F), lambda i: (0, 0))),
        out_shape=(jax.ShapeDtypeStruct((N, D), jnp.float32),
                   jax.ShapeDtypeStruct((NBLK, 8, D), jnp.float32),
                   jax.ShapeDtypeStruct((G, F), jnp.float32)),
    )(x, w1, batch3)


def _bn_coeffs(part, g, b):
    m = jnp.sum(part[:, 0, :], axis=0) * (1.0 / N)
    ex2 = jnp.sum(part[:, 1, :], axis=0) * (1.0 / N)
    v = ex2 - m * m
    scale = lax.rsqrt(v + EPS) * g
    return scale, b - m * scale


def _b_body(u_ref, part_ref, g1_ref, b1_ref, w12_ref, b12_ref, dinv_ref,
            tlo_ref, thi_ref):
    scale, shift = _bn_coeffs(part_ref[...], g1_ref[0], b1_ref[0])
    t = jnp.maximum(u_ref[...] * scale + shift, 0.0)
    hm = _dot(t, w12_ref[...]) + b12_ref[0]
    dv = dinv_ref[...]
    tlo_ref[...] = hm[:, :H] * dv
    thi_ref[...] = hm[:, H:] * dv


def _b_call(u, part, g1, b1, w12, b12, dinv):
    return pl.pallas_call(
        _b_body,
        grid=(NBLK,),
        in_specs=[
            pl.BlockSpec((BR, D), lambda i: (i, 0)),
            pl.BlockSpec((NBLK, 8, D), lambda i: (0, 0, 0)),
            pl.BlockSpec((8, D), lambda i: (0, 0)),
            pl.BlockSpec((8, D), lambda i: (0, 0)),
            pl.BlockSpec((D, D), lambda i: (0, 0)),
            pl.BlockSpec((8, D), lambda i: (0, 0)),
            pl.BlockSpec((BR, H), lambda i: (i, 0)),
        ],
        out_specs=(pl.BlockSpec((BR, H), lambda i: (i, 0)),
                   pl.BlockSpec((BR, H), lambda i: (i, 0))),
        out_shape=(jax.ShapeDtypeStruct((N, H), jnp.float32),
                   jax.ShapeDtypeStruct((N, H), jnp.float32)),
    )(u, part, g1, b1, w12, b12, dinv)


def _c_body(slo_ref, shi_ref, tlo_ref, thi_ref, dinv_ref,
            alo_ref, ahi_ref, part_ref):
    dv = dinv_ref[...]
    alo = dv * (slo_ref[...] + tlo_ref[...])
    ahi = dv * (shi_ref[...] + thi_ref[...])
    alo_ref[...] = alo
    ahi_ref[...] = ahi
    s = jnp.concatenate([jnp.sum(alo, 0), jnp.sum(ahi, 0)])
    q = jnp.concatenate([jnp.sum(alo * alo, 0), jnp.sum(ahi * ahi, 0)])
    part_ref[0] = jnp.concatenate(
        [s[None], q[None], jnp.zeros((6, D), jnp.float32)], axis=0)


def _c_call(slo, shi, tlo, thi, dinv):
    bs = pl.BlockSpec((BR, H), lambda i: (i, 0))
    return pl.pallas_call(
        _c_body,
        grid=(NBLK,),
        in_specs=[bs, bs, bs, bs, bs],
        out_specs=(bs, bs, pl.BlockSpec((1, 8, D), lambda i: (i, 0, 0))),
        out_shape=(jax.ShapeDtypeStruct((N, H), jnp.float32),
                   jax.ShapeDtypeStruct((N, H), jnp.float32),
                   jax.ShapeDtypeStruct((NBLK, 8, D), jnp.float32)),
    )(slo, shi, tlo, thi, dinv)


def _da_body(alo_ref, ahi_ref, part_ref, g2_ref, b2_ref, batch_ref, w1n_ref,
             pooled_ref, u_ref, parta_ref):
    i = pl.program_id(0)
    scale, shift = _bn_coeffs(part_ref[...], g2_ref[0], b2_ref[0])
    hlo = jnp.maximum(alo_ref[...] * scale[:H] + shift[:H], 0.0)
    hhi = jnp.maximum(ahi_ref[...] * scale[H:] + shift[H:], 0.0)
    b = batch_ref[0, 0]
    oh = (lax.broadcasted_iota(jnp.int32, (G, BR), 0) == b[None, :]
          ).astype(jnp.float32)

    @pl.when(i == 0)
    def _():
        pooled_ref[...] = jnp.zeros((G, D), jnp.float32)

    pooled_ref[...] += jnp.concatenate([_dot(oh, hlo), _dot(oh, hhi)], axis=1)
    if w1n_ref is not None:
        wn = w1n_ref[...]
        u = _dot(hlo, wn[:H, :]) + _dot(hhi, wn[H:, :])
        u_ref[...] = u
        parta_ref[0] = _stats_rows(u)


def _da_call(alo, ahi, part, g2, b2, batch3, w1n):
    bs = pl.BlockSpec((BR, H), lambda i: (i, 0))
    last = w1n is None
    in_specs = [
        bs, bs,
        pl.BlockSpec((NBLK, 8, D), lambda i: (0, 0, 0)),
        pl.BlockSpec((8, D), lambda i: (0, 0)),
        pl.BlockSpec((8, D), lambda i: (0, 0)),
        pl.BlockSpec((1, 1, BR), lambda i: (i, 0, 0)),
    ]
    args = [alo, ahi, part, g2, b2, batch3]
    out_specs = [pl.BlockSpec((G, D), lambda i: (0, 0))]
    out_shape = [jax.ShapeDtypeStruct((G, D), jnp.float32)]
    if last:
        body = functools.partial(_da_body_last)
        return pl.pallas_call(
            body, grid=(NBLK,), in_specs=in_specs,
            out_specs=out_specs[0], out_shape=out_shape[0])(*args)
    in_specs.append(pl.BlockSpec((D, D), lambda i: (0, 0)))
    args.append(w1n)
    out_specs += [pl.BlockSpec((BR, D), lambda i: (i, 0)),
                  pl.BlockSpec((1, 8, D), lambda i: (i, 0, 0))]
    out_shape += [jax.ShapeDtypeStruct((N, D), jnp.float32),
                  jax.ShapeDtypeStruct((NBLK, 8, D), jnp.float32)]
    return pl.pallas_call(
        _da_body, grid=(NBLK,), in_specs=in_specs,
        out_specs=tuple(out_specs), out_shape=tuple(out_shape))(*args)


def _da_body_last(alo_ref, ahi_ref, part_ref, g2_ref, b2_ref, batch_ref,
                  pooled_ref):
    _da_body(alo_ref, ahi_ref, part_ref, g2_ref, b2_ref, batch_ref, None,
             pooled_ref, None, None)


def _fc_body(p0_ref, p1_ref, p2_ref, p3_ref, p4_ref,
             w0_ref, w1_ref, w2_ref, w3_ref, w4_ref, b_ref, o_ref):
    acc = _dot(p0_ref[...], w0_ref[...])
    acc += _dot(p1_ref[...], w1_ref[...])
    acc += _dot(p2_ref[...], w2_ref[...])
    acc += _dot(p3_ref[...], w3_ref[...])
    acc += _dot(p4_ref[...], w4_ref[...])
    o_ref[...] = acc + b_ref[0]


def _fc_call(pools, ws, bsum):
    args = list(pools) + list(ws) + [bsum]
    return pl.pallas_call(
        _fc_body,
        in_specs=[pl.BlockSpec(a.shape, lambda: tuple(0 for _ in a.shape))
                  for a in args],
        out_specs=pl.BlockSpec((G, 128), lambda: (0, 0)),
        out_shape=jax.ShapeDtypeStruct((G, 128), jnp.float32),
    )(*args)


def kernel(x, params, edge_index, batch):
    f32 = jnp.float32
    row = edge_index[0].astype(jnp.int32)
    col = edge_index[1].astype(jnp.int32)
    npad = E_PAD - E
    pad_r = (jnp.arange(npad, dtype=jnp.int32) * 97) % N
    pad_c = N + (jnp.arange(npad, dtype=jnp.int32) % CH)
    row3d = jnp.concatenate([row, pad_r]).reshape(NS, NCHUNK, CH)
    col3d = jnp.concatenate([col, pad_c]).reshape(NS, NCHUNK, CH)
    batch3 = batch.astype(jnp.int32).reshape(NBLK, 1, BR)
    zrows = jnp.zeros((ZCH, H), f32)

    def pad8(v):
        return jnp.broadcast_to(v[None, :], (8, v.shape[0]))

    w2s = jnp.stack([params[f"l2W{i}"] for i in range(NUM_LAYERS)])
    gws = jnp.stack([params[f"gW{i}"] for i in range(NUM_LAYERS)])
    b2s = jnp.stack([pad8(params[f"l2b{i}"]) for i in range(NUM_LAYERS)])
    w12s, b12s = _prep(w2s, gws, b2s)

    ones = jnp.ones((CH, DEGW), f32)
    zeros16 = jnp.zeros((ZCH, DEGW), f32)
    degp16 = _sc_deg()(col3d, ones, zeros16)
    degp = jnp.broadcast_to(degp16[:, :, :1], (NC, N_ACC, H))

    u, part, pooled_x = _a0(x, params["l1W0"], batch3)
    dinv = _dinv_call(degp)
    pools = [pooled_x]
    for i in range(NUM_LAYERS):
        tlo, thi = _b_call(u, part, pad8(params[f"bn1g{i}"]),
                           pad8(params[f"bn1b{i}"]), w12s[i], b12s[i], dinv)
        slo, shi = _sc_scatter()(tlo, thi, row3d, col3d, zrows)
        alo, ahi, partc = _c_call(slo, shi, tlo, thi, dinv)
        g2 = pad8(params[f"bng{i}"])
        b2 = pad8(params[f"bnb{i}"])
        if i < NUM_LAYERS - 1:
            pooled_i, u, part = _da_call(alo, ahi, partc, g2, b2, batch3,
                                         params[f"l1W{i + 1}"])
        else:
            pooled_i = _da_call(alo, ahi, partc, g2, b2, batch3, None)
        pools.append(pooled_i)

    ws = []
    for i in range(NUM_LAYERS + 1):
        w = params[f"fcW{i}"]
        ws.append(jnp.zeros((w.shape[0], 128), f32).at[:, :OUT].set(w))
    bsum = sum(params[f"fcb{i}"] for i in range(NUM_LAYERS + 1))
    bpad = pad8(jnp.zeros((128,), f32).at[:OUT].set(bsum))
    out = _fc_call(pools, ws, bpad)
    return out[:, :OUT]


# speedup vs baseline: 1.0428x; 1.0147x over previous
import functools

import jax
import jax.numpy as jnp
from jax import lax
from jax.experimental import pallas as pl
from jax.experimental.pallas import tpu as pltpu
from jax.experimental.pallas import tpu_sc as plsc

N, E, F, D, OUT, G = 10000, 320000, 128, 256, 40, 64
NUM_LAYERS = 4
H = 128
NC, NS = 2, 16
CH = 64
EPT = 20480
NCHUNK = EPT // CH
E_PAD = EPT * NS
GCH = NCHUNK // 5
NGRP = NCHUNK // GCH
NB = 4
N_ACC = 10240
ZROWS = N_ACC // NS
BR = 1000
NBLK = N // BR
EPS = 1e-5

@functools.cache
def _sc_mesh():
    return plsc.VectorSubcoreMesh(core_axis_name="c", subcore_axis_name="s",
                                  num_cores=NC, num_subcores=NS)


HW = 16
ZCH = 64


DEGW = HW


def _sc_deg_body(col3d, ones_hbm, zeros_hbm, out,
                 col_v, ones_v, zb_v, acc):
    cid = lax.axis_index("c")
    sid = lax.axis_index("s")
    pltpu.sync_copy(zeros_hbm, zb_v)
    pltpu.sync_copy(ones_hbm, ones_v)

    def zchunk(t, carry):
        pltpu.sync_copy(zb_v, acc.at[pl.ds(sid * ZROWS + t * ZCH, ZCH)])
        return carry

    lax.fori_loop(0, ZROWS // ZCH, zchunk, 0)
    plsc.subcore_barrier()
    base = cid * (GCH // 2)

    def group(g, carry):
        pltpu.sync_copy(col3d.at[sid, pl.ds(g * GCH, GCH)], col_v)

        def step(k, carry2):
            pltpu.sync_copy(ones_v, acc.at[col_v.at[base + k]], add=True)
            return carry2

        lax.fori_loop(0, GCH // 2, step, 0)
        return carry

    lax.fori_loop(0, NGRP, group, 0)
    plsc.subcore_barrier()
    pltpu.sync_copy(acc.at[pl.ds(sid * ZROWS, ZROWS)],
                    out.at[cid, pl.ds(sid * ZROWS, ZROWS)])


@functools.cache
def _sc_deg():
    return pl.kernel(
        _sc_deg_body,
        out_type=jax.ShapeDtypeStruct((NC, N_ACC, DEGW), jnp.float32),
        mesh=_sc_mesh(),
        scratch_types=[
            pltpu.VMEM((GCH, CH), jnp.int32),
            pltpu.VMEM((CH, DEGW), jnp.float32),
            pltpu.VMEM((ZCH, DEGW), jnp.float32),
            pltpu.VMEM_SHARED((N_ACC, DEGW), jnp.float32),
        ],
    )


def _sc_scatter_body(t_lo, t_hi, row3d, col3d, zrows_hbm, s_lo, s_hi,
                     row_v, col_v, acc, ssem, *rest):
    cid = lax.axis_index("c")
    sid = lax.axis_index("s")
    bufs = rest[:NB]
    gsems = rest[NB:]
    zb_v = bufs[0]
    pltpu.sync_copy(zrows_hbm, zb_v)

    def zchunk(t, carry):
        pltpu.sync_copy(zb_v, acc.at[pl.ds(sid * ZROWS + t * ZCH, ZCH)])
        return carry

    lax.fori_loop(0, ZROWS // ZCH, zchunk, 0)
    plsc.subcore_barrier()

    def run(table):
        def group(g, carry):
            pltpu.sync_copy(row3d.at[sid, pl.ds(g * GCH, GCH)], row_v)
            pltpu.sync_copy(col3d.at[sid, pl.ds(g * GCH, GCH)], col_v)
            for b in range(NB - 1):
                pltpu.async_copy(table.at[row_v.at[b]], bufs[b], gsems[b])

            def quad(m, carry2):
                for b in range(NB):
                    jj = NB * m + b
                    nxt = jj + NB - 1
                    bn = (b + NB - 1) % NB
                    pltpu.make_async_copy(table.at[row_v.at[jj]], bufs[b],
                                          gsems[b]).wait()
                    pltpu.async_copy(bufs[b], acc.at[col_v.at[jj]], ssem,
                                     add=True)

                    @pl.when(jnp.logical_and(jj >= 1, nxt <= GCH - 1))
                    def _():
                        pltpu.make_async_copy(
                            bufs[0], acc.at[col_v.at[0]], ssem).wait()

                    @pl.when(nxt <= GCH - 1)
                    def _():
                        pltpu.async_copy(table.at[row_v.at[nxt]], bufs[bn],
                                         gsems[bn])
                return carry2

            lax.fori_loop(0, GCH // NB, quad, 0)
            for _ in range(NB):
                pltpu.make_async_copy(bufs[0], acc.at[col_v.at[0]],
                                      ssem).wait()
            return carry

        lax.fori_loop(0, NGRP, group, 0)

    @pl.when(cid == 0)
    def _():
        run(t_lo)

    @pl.when(cid == 1)
    def _():
        run(t_hi)

    plsc.subcore_barrier()

    @pl.when(cid == 0)
    def _():
        pltpu.sync_copy(acc.at[pl.ds(sid * ZROWS, ZROWS)],
                        s_lo.at[pl.ds(sid * ZROWS, ZROWS)])

    @pl.when(cid == 1)
    def _():
        pltpu.sync_copy(acc.at[pl.ds(sid * ZROWS, ZROWS)],
                        s_hi.at[pl.ds(sid * ZROWS, ZROWS)])


@functools.cache
def _sc_scatter():
    return pl.kernel(
        _sc_scatter_body,
        out_type=(jax.ShapeDtypeStruct((N_ACC, H), jnp.float32),
                  jax.ShapeDtypeStruct((N_ACC, H), jnp.float32)),
        mesh=_sc_mesh(),
        scratch_types=(
            [pltpu.VMEM((GCH, CH), jnp.int32),
             pltpu.VMEM((GCH, CH), jnp.int32),
             pltpu.VMEM_SHARED((N_ACC, H), jnp.float32),
             pltpu.SemaphoreType.DMA]
            + [pltpu.VMEM((CH, H), jnp.float32)] * NB
            + [pltpu.SemaphoreType.DMA] * NB),
    )


def _dot(a, b):
    return jnp.dot(a, b, preferred_element_type=jnp.float32)


def _prep_body(w2_ref, gw_ref, b2_ref, w12_ref, b12_ref):
    w12_ref[0] = _dot(w2_ref[0], gw_ref[0])
    b12_ref[0] = _dot(b2_ref[0], gw_ref[0])


def _prep(w2s, gws, b2s):
    return pl.pallas_call(
        _prep_body,
        grid=(NUM_LAYERS,),
        in_specs=[
            pl.BlockSpec((1, D, D), lambda i: (i, 0, 0)),
            pl.BlockSpec((1, D, D), lambda i: (i, 0, 0)),
            pl.BlockSpec((1, 8, D), lambda i: (i, 0, 0)),
        ],
        out_specs=(pl.BlockSpec((1, D, D), lambda i: (i, 0, 0)),
                   pl.BlockSpec((1, 8, D), lambda i: (i, 0, 0))),
        out_shape=(jax.ShapeDtypeStruct((NUM_LAYERS, D, D), jnp.float32),
                   jax.ShapeDtypeStruct((NUM_LAYERS, 8, D), jnp.float32)),
    )(w2s, gws, b2s)


def _stats_rows(h):
    s = jnp.sum(h, axis=0)
    q = jnp.sum(h * h, axis=0)
    return jnp.concatenate(
        [s[None], q[None], jnp.zeros((6, s.shape[0]), jnp.float32)], axis=0)


def _dinv_body(degp_ref, dinv_ref):
    d = lax.rsqrt(degp_ref[0] + degp_ref[1] + 1.0)
    dinv_ref[...] = jnp.broadcast_to(d[:, :1], (BR, H))


def _dinv_call(degp):
    return pl.pallas_call(
        _dinv_body,
        grid=(NBLK,),
        in_specs=[pl.BlockSpec((NC, BR, DEGW), lambda i: (0, i, 0))],
        out_specs=pl.BlockSpec((BR, H), lambda i: (i, 0)),
        out_shape=jax.ShapeDtypeStruct((N, H), jnp.float32),
    )(degp)


def _a0_body(x_ref, w1_ref, batch_ref,
             u_ref, part_ref, pooled_ref):
    i = pl.program_id(0)
    xb = x_ref[...]
    u = _dot(xb, w1_ref[...])
    u_ref[...] = u
    part_ref[0] = _stats_rows(u)
    b = batch_ref[0, 0]
    oh = (lax.broadcasted_iota(jnp.int32, (G, BR), 0) == b[None, :]
          ).astype(jnp.float32)

    @pl.when(i == 0)
    def _():
        pooled_ref[...] = jnp.zeros((G, F), jnp.float32)

    pooled_ref[...] += _dot(oh, xb)


def _a0(x, w1, batch3):
    return pl.pallas_call(
        _a0_body,
        grid=(NBLK,),
        in_specs=[
            pl.BlockSpec((BR, F), lambda i: (i, 0)),
            pl.BlockSpec((F, D), lambda i: (0, 0)),
            pl.BlockSpec((1, 1, BR), lambda i: (i, 0, 0)),
        ],
        out_specs=(pl.BlockSpec((BR, D), lambda i: (i, 0)),
                   pl.BlockSpec((1, 8, D), lambda i: (i, 0, 0)),
                   pl.BlockSpec((G, F), lambda i: (0, 0))),
        out_shape=(jax.ShapeDtypeStruct((N, D), jnp.float32),
                   jax.ShapeDtypeStruct((NBLK, 8, D), jnp.float32),
                   jax.ShapeDtypeStruct((G, F), jnp.float32)),
    )(x, w1, batch3)


def _bn_coeffs(part, g, b):
    m = jnp.sum(part[:, 0, :], axis=0) * (1.0 / N)
    ex2 = jnp.sum(part[:, 1, :], axis=0) * (1.0 / N)
    v = ex2 - m * m
    scale = lax.rsqrt(v + EPS) * g
    return scale, b - m * scale


def _b_body(u_ref, part_ref, g1_ref, b1_ref, w12_ref, b12_ref, dinv_ref,
            tlo_ref, thi_ref):
    scale, shift = _bn_coeffs(part_ref[...], g1_ref[0], b1_ref[0])
    t = jnp.maximum(u_ref[...] * scale + shift, 0.0)
    hm = _dot(t, w12_ref[...]) + b12_ref[0]
    dv = dinv_ref[...]
    tlo_ref[...] = hm[:, :H] * dv
    thi_ref[...] = hm[:, H:] * dv


def _b_call(u, part, g1, b1, w12, b12, dinv):
    return pl.pallas_call(
        _b_body,
        grid=(NBLK,),
        in_specs=[
            pl.BlockSpec((BR, D), lambda i: (i, 0)),
            pl.BlockSpec((NBLK, 8, D), lambda i: (0, 0, 0)),
            pl.BlockSpec((8, D), lambda i: (0, 0)),
            pl.BlockSpec((8, D), lambda i: (0, 0)),
            pl.BlockSpec((D, D), lambda i: (0, 0)),
            pl.BlockSpec((8, D), lambda i: (0, 0)),
            pl.BlockSpec((BR, H), lambda i: (i, 0)),
        ],
        out_specs=(pl.BlockSpec((BR, H), lambda i: (i, 0)),
                   pl.BlockSpec((BR, H), lambda i: (i, 0))),
        out_shape=(jax.ShapeDtypeStruct((N, H), jnp.float32),
                   jax.ShapeDtypeStruct((N, H), jnp.float32)),
    )(u, part, g1, b1, w12, b12, dinv)


def _c_body(slo_ref, shi_ref, tlo_ref, thi_ref, dinv_ref,
            alo_ref, ahi_ref, part_ref):
    dv = dinv_ref[...]
    alo = dv * (slo_ref[...] + tlo_ref[...])
    ahi = dv * (shi_ref[...] + thi_ref[...])
    alo_ref[...] = alo
    ahi_ref[...] = ahi
    s = jnp.concatenate([jnp.sum(alo, 0), jnp.sum(ahi, 0)])
    q = jnp.concatenate([jnp.sum(alo * alo, 0), jnp.sum(ahi * ahi, 0)])
    part_ref[0] = jnp.concatenate(
        [s[None], q[None], jnp.zeros((6, D), jnp.float32)], axis=0)


def _c_call(slo, shi, tlo, thi, dinv):
    bs = pl.BlockSpec((BR, H), lambda i: (i, 0))
    return pl.pallas_call(
        _c_body,
        grid=(NBLK,),
        in_specs=[bs, bs, bs, bs, bs],
        out_specs=(bs, bs, pl.BlockSpec((1, 8, D), lambda i: (i, 0, 0))),
        out_shape=(jax.ShapeDtypeStruct((N, H), jnp.float32),
                   jax.ShapeDtypeStruct((N, H), jnp.float32),
                   jax.ShapeDtypeStruct((NBLK, 8, D), jnp.float32)),
    )(slo, shi, tlo, thi, dinv)


def _da_body(alo_ref, ahi_ref, part_ref, g2_ref, b2_ref, batch_ref, w1n_ref,
             pooled_ref, u_ref, parta_ref):
    i = pl.program_id(0)
    scale, shift = _bn_coeffs(part_ref[...], g2_ref[0], b2_ref[0])
    hlo = jnp.maximum(alo_ref[...] * scale[:H] + shift[:H], 0.0)
    hhi = jnp.maximum(ahi_ref[...] * scale[H:] + shift[H:], 0.0)
    b = batch_ref[0, 0]
    oh = (lax.broadcasted_iota(jnp.int32, (G, BR), 0) == b[None, :]
          ).astype(jnp.float32)

    @pl.when(i == 0)
    def _():
        pooled_ref[...] = jnp.zeros((G, D), jnp.float32)

    pooled_ref[...] += jnp.concatenate([_dot(oh, hlo), _dot(oh, hhi)], axis=1)
    if w1n_ref is not None:
        wn = w1n_ref[...]
        u = _dot(hlo, wn[:H, :]) + _dot(hhi, wn[H:, :])
        u_ref[...] = u
        parta_ref[0] = _stats_rows(u)


def _da_call(alo, ahi, part, g2, b2, batch3, w1n):
    bs = pl.BlockSpec((BR, H), lambda i: (i, 0))
    last = w1n is None
    in_specs = [
        bs, bs,
        pl.BlockSpec((NBLK, 8, D), lambda i: (0, 0, 0)),
        pl.BlockSpec((8, D), lambda i: (0, 0)),
        pl.BlockSpec((8, D), lambda i: (0, 0)),
        pl.BlockSpec((1, 1, BR), lambda i: (i, 0, 0)),
    ]
    args = [alo, ahi, part, g2, b2, batch3]
    out_specs = [pl.BlockSpec((G, D), lambda i: (0, 0))]
    out_shape = [jax.ShapeDtypeStruct((G, D), jnp.float32)]
    if last:
        body = functools.partial(_da_body_last)
        return pl.pallas_call(
            body, grid=(NBLK,), in_specs=in_specs,
            out_specs=out_specs[0], out_shape=out_shape[0])(*args)
    in_specs.append(pl.BlockSpec((D, D), lambda i: (0, 0)))
    args.append(w1n)
    out_specs += [pl.BlockSpec((BR, D), lambda i: (i, 0)),
                  pl.BlockSpec((1, 8, D), lambda i: (i, 0, 0))]
    out_shape += [jax.ShapeDtypeStruct((N, D), jnp.float32),
                  jax.ShapeDtypeStruct((NBLK, 8, D), jnp.float32)]
    return pl.pallas_call(
        _da_body, grid=(NBLK,), in_specs=in_specs,
        out_specs=tuple(out_specs), out_shape=tuple(out_shape))(*args)


def _da_body_last(alo_ref, ahi_ref, part_ref, g2_ref, b2_ref, batch_ref,
                  pooled_ref):
    _da_body(alo_ref, ahi_ref, part_ref, g2_ref, b2_ref, batch_ref, None,
             pooled_ref, None, None)


def _fc_body(p0_ref, p1_ref, p2_ref, p3_ref, p4_ref,
             w0_ref, w1_ref, w2_ref, w3_ref, w4_ref, b_ref, o_ref):
    acc = _dot(p0_ref[...], w0_ref[...])
    acc += _dot(p1_ref[...], w1_ref[...])
    acc += _dot(p2_ref[...], w2_ref[...])
    acc += _dot(p3_ref[...], w3_ref[...])
    acc += _dot(p4_ref[...], w4_ref[...])
    o_ref[...] = acc + b_ref[0]


def _fc_call(pools, ws, bsum):
    args = list(pools) + list(ws) + [bsum]
    return pl.pallas_call(
        _fc_body,
        in_specs=[pl.BlockSpec(a.shape, lambda: tuple(0 for _ in a.shape))
                  for a in args],
        out_specs=pl.BlockSpec((G, 128), lambda: (0, 0)),
        out_shape=jax.ShapeDtypeStruct((G, 128), jnp.float32),
    )(*args)


def kernel(x, params, edge_index, batch):
    f32 = jnp.float32
    row = edge_index[0].astype(jnp.int32)
    col = edge_index[1].astype(jnp.int32)
    npad = E_PAD - E
    pad_r = (jnp.arange(npad, dtype=jnp.int32) * 97) % N
    pad_c = N + (jnp.arange(npad, dtype=jnp.int32) % CH)
    row3d = jnp.concatenate([row, pad_r]).reshape(NS, NCHUNK, CH)
    col3d = jnp.concatenate([col, pad_c]).reshape(NS, NCHUNK, CH)
    batch3 = batch.astype(jnp.int32).reshape(NBLK, 1, BR)
    zrows = jnp.zeros((ZCH, H), f32)

    def pad8(v):
        return jnp.broadcast_to(v[None, :], (8, v.shape[0]))

    w2s = jnp.stack([params[f"l2W{i}"] for i in range(NUM_LAYERS)])
    gws = jnp.stack([params[f"gW{i}"] for i in range(NUM_LAYERS)])
    b2s = jnp.stack([pad8(params[f"l2b{i}"]) for i in range(NUM_LAYERS)])
    w12s, b12s = _prep(w2s, gws, b2s)

    ones = jnp.ones((CH, DEGW), f32)
    zeros16 = jnp.zeros((ZCH, DEGW), f32)
    degp = _sc_deg()(col3d, ones, zeros16)

    u, part, pooled_x = _a0(x, params["l1W0"], batch3)
    dinv = _dinv_call(degp)
    pools = [pooled_x]
    for i in range(NUM_LAYERS):
        tlo, thi = _b_call(u, part, pad8(params[f"bn1g{i}"]),
                           pad8(params[f"bn1b{i}"]), w12s[i], b12s[i], dinv)
        slo, shi = _sc_scatter()(tlo, thi, row3d, col3d, zrows)
        alo, ahi, partc = _c_call(slo, shi, tlo, thi, dinv)
        g2 = pad8(params[f"bng{i}"])
        b2 = pad8(params[f"bnb{i}"])
        if i < NUM_LAYERS - 1:
            pooled_i, u, part = _da_call(alo, ahi, partc, g2, b2, batch3,
                                         params[f"l1W{i + 1}"])
        else:
            pooled_i = _da_call(alo, ahi, partc, g2, b2, batch3, None)
        pools.append(pooled_i)

    ws = []
    for i in range(NUM_LAYERS + 1):
        w = params[f"fcW{i}"]
        ws.append(jnp.zeros((w.shape[0], 128), f32).at[:, :OUT].set(w))
    bsum = sum(params[f"fcb{i}"] for i in range(NUM_LAYERS + 1))
    bpad = pad8(jnp.zeros((128,), f32).at[:OUT].set(bsum))
    out = _fc_call(pools, ws, bpad)
    return out[:, :OUT]
